# baseline (speedup 1.0000x reference)
import functools
import jax
import jax.numpy as jnp
from jax import lax
from jax.experimental import pallas as pl
from jax.experimental.pallas import tpu as pltpu
from jax.experimental.pallas import tpu_sc as plsc

USER_N = 5000
ITEM_N = 5000
N_NODES = USER_N + ITEM_N
N_EDGES = 160000
EMB = 256
N_LAYERS = 3

CPT = 8
CPP = 4
NT = 32
SLAB = N_NODES * CPT
HSLAB = N_NODES * CPP
CHUNK = 1600
NCHUNK = N_EDGES // CHUNK
CELEM = CHUNK * CPP
ZELEM = 4096


def _sc_body(ego0_hbm, src_hbm, dst_hbm, w_hbm,
             l1_hbm, l2_hbm, out_hbm,
             table, srcb, dstb, wb, sidxb, updb, zbuf,
             acc, esem, ssem):
    c = lax.axis_index("c")
    s = lax.axis_index("s")
    t = c * 16 + s
    aoff = s * HSLAB
    iota = lax.iota(jnp.int32, 16)
    iota4 = iota * 4

    def _zb(r, carry):
        zbuf[pl.ds(r * 16, 16)] = jnp.zeros((16,), jnp.float32)
        return carry
    lax.fori_loop(0, ZELEM // 16, _zb, 0)

    def zero_acc():
        nfull = HSLAB // ZELEM
        for j in range(nfull):
            pltpu.sync_copy(zbuf, acc.at[pl.ds(aoff + j * ZELEM, ZELEM)])
        rem = HSLAB - nfull * ZELEM
        if rem:
            pltpu.sync_copy(zbuf.at[pl.ds(0, rem)],
                            acc.at[pl.ds(aoff + HSLAB - rem, rem)])

    def fire_edges(g, p):
        off = g * CHUNK
        pltpu.make_async_copy(src_hbm.at[pl.ds(off, CHUNK)], srcb.at[p],
                              esem.at[p]).start()
        pltpu.make_async_copy(dst_hbm.at[pl.ds(off, CHUNK)], dstb.at[p],
                              esem.at[p]).start()
        pltpu.make_async_copy(w_hbm.at[pl.ds(off, CHUNK)], wb.at[p],
                              esem.at[p]).start()

    def wait_edges(g, p):
        off = g * CHUNK
        pltpu.make_async_copy(src_hbm.at[pl.ds(off, CHUNK)], srcb.at[p],
                              esem.at[p]).wait()
        pltpu.make_async_copy(dst_hbm.at[pl.ds(off, CHUNK)], dstb.at[p],
                              esem.at[p]).wait()
        pltpu.make_async_copy(w_hbm.at[pl.ds(off, CHUNK)], wb.at[p],
                              esem.at[p]).wait()

    def fire_scatter(p):
        pltpu.make_async_copy(updb.at[p], acc.at[sidxb.at[p]],
                              ssem.at[p]).start(add=True)

    def wait_scatter(p):
        pltpu.make_async_copy(updb.at[p], acc.at[sidxb.at[p]],
                              ssem.at[p]).wait()

    def compute(p):
        updb_p = updb.at[p]
        sidx_p = sidxb.at[p]

        def body(r, carry):
            for b in range(2):
                base = r * 32 + b * 16
                src16 = srcb[p, pl.ds(base, 16)]
                dst16 = dstb[p, pl.ds(base, 16)]
                w16 = wb[p, pl.ds(base, 16)]
                src4 = src16 * 4
                didx4 = dst16 * 4 + aoff
                pos0 = base * 4 + iota4
                for cc in range(CPP):
                    gv = plsc.load_gather(table, [src4 + cc])
                    plsc.store_scatter(updb_p, [pos0 + cc], gv * w16)
                    plsc.store_scatter(sidx_p, [pos0 + cc], didx4 + cc)
            return carry
        lax.fori_loop(0, CHUNK // 32, body, 0)

    def unit(g, p, fire_next, wait_sct):
        wait_edges(g, p)
        if fire_next:
            fire_edges(g + 1, 1 - p)
        if wait_sct:
            wait_scatter(p)
        compute(p)
        fire_scatter(p)

    def sweep():
        zero_acc()
        fire_edges(0, 0)
        unit(0, 0, True, False)
        unit(1, 1, True, False)

        def two(i, carry):
            unit(2 * i, 0, True, True)
            unit(2 * i + 1, 1, True, True)
            return carry
        lax.fori_loop(1, NCHUNK // 2 - 1, two, 0)
        unit(NCHUNK - 2, 0, True, True)
        unit(NCHUNK - 1, 1, False, True)
        wait_scatter(0)
        wait_scatter(1)

    for q in range(2):
        goff = t * SLAB + q * HSLAB
        pltpu.sync_copy(ego0_hbm.at[pl.ds(goff, HSLAB)], table)

        for layer in range(N_LAYERS):
            sweep()
            pltpu.sync_copy(acc.at[pl.ds(aoff, HSLAB)], table)
            if layer == 0:
                pltpu.sync_copy(table, l1_hbm.at[pl.ds(goff, HSLAB)])
            elif layer == 1:
                pltpu.sync_copy(table, l2_hbm.at[pl.ds(goff, HSLAB)])

        FE = 3200
        b_e0 = updb.at[0, pl.ds(0, FE)]
        b_l1 = updb.at[0, pl.ds(FE, FE)]
        b_l2 = updb.at[1, pl.ds(0, FE)]
        b_out = updb.at[1, pl.ds(FE, FE)]
        for j in range(HSLAB // FE):
            e0 = j * FE
            pltpu.sync_copy(ego0_hbm.at[pl.ds(goff + e0, FE)], b_e0)
            pltpu.sync_copy(l1_hbm.at[pl.ds(goff + e0, FE)], b_l1)
            pltpu.sync_copy(l2_hbm.at[pl.ds(goff + e0, FE)], b_l2)

            def fbody(r, carry):
                sl = pl.ds(r * 16, 16)
                v = (b_e0[sl] + b_l1[sl] + b_l2[sl]
                     + table[pl.ds(e0 + r * 16, 16)])
                b_out[sl] = v * 0.25
                return carry
            lax.fori_loop(0, FE // 16, fbody, 0)
            pltpu.sync_copy(b_out, out_hbm.at[pl.ds(goff + e0, FE)])
        REM = HSLAB - (HSLAB // FE) * FE
        if REM:
            e0 = (HSLAB // FE) * FE
            pltpu.sync_copy(ego0_hbm.at[pl.ds(goff + e0, REM)],
                            updb.at[0, pl.ds(0, REM)])
            pltpu.sync_copy(l1_hbm.at[pl.ds(goff + e0, REM)],
                            updb.at[0, pl.ds(FE, REM)])
            pltpu.sync_copy(l2_hbm.at[pl.ds(goff + e0, REM)],
                            updb.at[1, pl.ds(0, REM)])

            def rbody(r, carry):
                sl = pl.ds(r * 16, 16)
                v = (b_e0[sl] + b_l1[sl] + b_l2[sl]
                     + table[pl.ds(e0 + r * 16, 16)])
                b_out[sl] = v * 0.25
                return carry
            lax.fori_loop(0, REM // 16, rbody, 0)
            pltpu.sync_copy(updb.at[1, pl.ds(FE, REM)],
                            out_hbm.at[pl.ds(goff + e0, REM)])


@jax.jit
def _run(ego0_t, src, dst, w):
    f32 = jnp.float32
    kfn = functools.partial(
        pl.kernel,
        out_type=[
            jax.ShapeDtypeStruct((NT * SLAB,), f32),
            jax.ShapeDtypeStruct((NT * SLAB,), f32),
            jax.ShapeDtypeStruct((NT * SLAB,), f32),
        ],
        scratch_types=[
            pltpu.VMEM((HSLAB,), f32),
            pltpu.VMEM((2, CHUNK), jnp.int32),
            pltpu.VMEM((2, CHUNK), jnp.int32),
            pltpu.VMEM((2, CHUNK), f32),
            pltpu.VMEM((2, CELEM), jnp.int32),
            pltpu.VMEM((2, CELEM), f32),
            pltpu.VMEM((ZELEM,), f32),
            pltpu.VMEM_SHARED((16 * HSLAB,), f32),
            pltpu.SemaphoreType.DMA((2,)),
            pltpu.SemaphoreType.DMA((2,)),
        ],
        mesh=plsc.VectorSubcoreMesh(core_axis_name="c", subcore_axis_name="s"),
        compiler_params=pltpu.CompilerParams(
            needs_layout_passes=False, use_tc_tiling_on_sc=False),
    )(_sc_body)
    _l1, _l2, out = kfn(ego0_t, src, dst, w)
    return (out.reshape(NT, 2, N_NODES, CPP).transpose(2, 0, 1, 3)
            .reshape(N_NODES, EMB))


def kernel(user_emb, item_emb, edge_index, edge_weight):
    ego = jnp.concatenate([user_emb, item_emb], axis=0)
    ego0_t = ego.reshape(N_NODES, NT, 2, CPP).transpose(1, 2, 0, 3).reshape(-1)
    out = _run(ego0_t, edge_index[0], edge_index[1], edge_weight)
    return (out[:USER_N], out[USER_N:])
